# Initial kernel scaffold; baseline (speedup 1.0000x reference)
import jax
import jax.numpy as jnp
from jax.experimental import pallas as pl


def kernel(x, edge_index, node_index, W1, a1_src, a1_dst, b1, W2, a2_src, a2_dst, b2):
    raise NotImplementedError("write your pallas kernel here")


# speedup vs baseline: 46.0413x; 46.0413x over previous
import functools

import jax
import jax.numpy as jnp
from jax import lax
from jax.experimental import pallas as pl
from jax.experimental.pallas import tpu as pltpu
from jax.experimental.pallas import tpu_sc as plsc

NCORE = 2
NSUB = 16
NW = NCORE * NSUB

_GDN = lax.GatherDimensionNumbers(
    offset_dims=(), collapsed_slice_dims=(0,), start_index_map=(0,))


def _lane_gather(v16, idx16):
    return lax.gather(v16, idx16[:, None], dimension_numbers=_GDN,
                      slice_sizes=(1,),
                      mode=lax.GatherScatterMode.PROMISE_IN_BOUNDS)


def _tc1_body(x_ref, w1_ref, acatA_ref, acatB_ref, h_ref, tA_ref, tB_ref):
    h = jnp.dot(x_ref[...], w1_ref[...], preferred_element_type=jnp.float32)
    h_ref[...] = h
    tA_ref[...] = jnp.dot(h, acatA_ref[...], preferred_element_type=jnp.float32)
    tB_ref[...] = jnp.dot(h, acatB_ref[...], preferred_element_type=jnp.float32)


def _tc2_body(op0_ref, op1_ref, dp0_ref, dp1_ref, r16_ref, b1_ref, w2_ref,
              a2A_ref, a2B_ref, h2_ref, t2A_ref, t2B_ref):
    den = jnp.dot(dp0_ref[...] + dp1_ref[...], r16_ref[...],
                  preferred_element_type=jnp.float32)
    out1 = (op0_ref[...] + op1_ref[...]) / (den + 1e-9) + b1_ref[...]
    hmid = jnp.where(out1 > 0, out1, jnp.exp(jnp.minimum(out1, 0.0)) - 1.0)
    h2 = jnp.dot(hmid, w2_ref[...], preferred_element_type=jnp.float32)
    h2_ref[...] = h2
    t2A_ref[...] = jnp.dot(h2, a2A_ref[...], preferred_element_type=jnp.float32)
    t2B_ref[...] = jnp.dot(h2, a2B_ref[...], preferred_element_type=jnp.float32)


def _tc3_body(mp0_ref, mp1_ref, ep0_ref, ep1_ref, r2_ref, b2_ref, out_ref):
    den = jnp.dot(ep0_ref[...] + ep1_ref[...], r2_ref[...],
                  preferred_element_type=jnp.float32)
    out_ref[...] = (mp0_ref[...] + mp1_ref[...]) / (den + 1e-9) + b2_ref[...]


def _make_sc_edge_kernel(n_pad, n_edges, feat, chunk, heads):
    ept = n_edges // NW
    nchunk = ept // chunk
    rows_per_sub = n_pad // NSUB
    nheads_blk = feat // 16

    mesh = plsc.VectorSubcoreMesh(core_axis_name="c", subcore_axis_name="s")

    @functools.partial(
        pl.kernel,
        mesh=mesh,
        compiler_params=pltpu.CompilerParams(use_tc_tiling_on_sc=False),
        out_type=[
            jax.ShapeDtypeStruct((NCORE * n_pad, feat), jnp.float32),
            jax.ShapeDtypeStruct((NCORE * n_pad, 16), jnp.float32),
        ],
        scratch_types=[
            pltpu.VMEM((chunk,), jnp.int32),
            pltpu.VMEM((chunk,), jnp.int32),
            pltpu.VMEM((chunk, 16), jnp.float32),
            pltpu.VMEM((chunk, 16), jnp.float32),
            pltpu.VMEM((chunk, feat), jnp.float32),
            pltpu.VMEM((chunk, 16), jnp.float32),
            pltpu.VMEM_SHARED((n_pad, feat), jnp.float32),
            pltpu.VMEM_SHARED((n_pad, 16), jnp.float32),
        ],
    )
    def sc_kernel(src_hbm, dst_hbm, tabA_hbm, tabB_hbm, h_hbm, zf_hbm, z16_hbm,
                  outp_hbm, denp_hbm,
                  sidx, didx, tabs, tabd, hrows, exb, out_sh, den_sh):
        c = lax.axis_index("c")
        s = lax.axis_index("s")
        wid = c * NSUB + s
        rbase = s * rows_per_sub

        pltpu.sync_copy(zf_hbm.at[pl.ds(rbase, rows_per_sub)],
                        out_sh.at[pl.ds(rbase, rows_per_sub)])
        pltpu.sync_copy(z16_hbm.at[pl.ds(rbase, rows_per_sub)],
                        den_sh.at[pl.ds(rbase, rows_per_sub)])
        plsc.subcore_barrier()

        @pl.loop(0, nchunk)
        def _chunk(k):
            base = wid * ept + k * chunk
            pltpu.sync_copy(src_hbm.at[pl.ds(base, chunk)], sidx)
            pltpu.sync_copy(dst_hbm.at[pl.ds(base, chunk)], didx)
            pltpu.sync_copy(tabA_hbm.at[sidx], tabs)
            pltpu.sync_copy(tabB_hbm.at[didx], tabd)
            pltpu.sync_copy(h_hbm.at[sidx], hrows)

            @pl.loop(0, chunk)
            def _edge(j):
                ea = tabs[j, :]
                eb = tabd[j, :]
                e = ea + eb
                e = jnp.maximum(e, 0.2 * e)
                ex = jnp.exp(e)
                exb[j, :] = ex
                for h in range(heads):
                    idx = jnp.full((16,), h, dtype=jnp.int32)
                    bc = _lane_gather(ex, idx)
                    if heads == 1:
                        for fblk in range(nheads_blk):
                            slc = (j, pl.ds(fblk * 16, 16))
                            hrows[slc] = hrows[slc] * bc
                    else:
                        slc = (j, pl.ds(h * 16, 16))
                        hrows[slc] = hrows[slc] * bc

            pltpu.sync_copy(exb, den_sh.at[didx], add=True)
            pltpu.sync_copy(hrows, out_sh.at[didx], add=True)

        plsc.subcore_barrier()
        pltpu.sync_copy(out_sh.at[pl.ds(rbase, rows_per_sub)],
                        outp_hbm.at[pl.ds(c * n_pad + rbase, rows_per_sub)])
        pltpu.sync_copy(den_sh.at[pl.ds(rbase, rows_per_sub)],
                        denp_hbm.at[pl.ds(c * n_pad + rbase, rows_per_sub)])

    return sc_kernel


def _make_sc_take_kernel(n_nodes, feat, nb):
    rows_per_tile = nb // NW
    mesh = plsc.VectorSubcoreMesh(core_axis_name="c", subcore_axis_name="s")

    @functools.partial(
        pl.kernel,
        mesh=mesh,
        compiler_params=pltpu.CompilerParams(use_tc_tiling_on_sc=False),
        out_type=jax.ShapeDtypeStruct((nb, feat), jnp.float32),
        scratch_types=[
            pltpu.VMEM((rows_per_tile,), jnp.int32),
            pltpu.VMEM((rows_per_tile, feat), jnp.float32),
        ],
    )
    def take_kernel(tab_hbm, idx_hbm, out_hbm, idxv, rowsv):
        c = lax.axis_index("c")
        s = lax.axis_index("s")
        wid = c * NSUB + s
        base = wid * rows_per_tile
        pltpu.sync_copy(idx_hbm.at[pl.ds(base, rows_per_tile)], idxv)
        pltpu.sync_copy(tab_hbm.at[idxv], rowsv)
        pltpu.sync_copy(rowsv, out_hbm.at[pl.ds(base, rows_per_tile)])

    return take_kernel


@jax.jit
def kernel(x, edge_index, node_index, W1, a1_src, a1_dst, b1,
           W2, a2_src, a2_dst, b2):
    n, f_in = x.shape
    e = edge_index.shape[1]
    heads, hid = a1_src.shape
    out_f = W2.shape[1]
    nb = node_index.shape[0]
    hdim = heads * hid

    src = edge_index[0]
    dst = edge_index[1]

    eyeh = jnp.eye(heads, dtype=jnp.float32)
    A1s = (eyeh[:, None, :] * a1_src[:, :, None]).reshape(hdim, heads)
    A1d = (eyeh[:, None, :] * a1_dst[:, :, None]).reshape(hdim, heads)
    acatA = jnp.concatenate([A1s, A1d], axis=1)
    acatB = jnp.concatenate([A1d, A1s], axis=1)
    pad2 = jnp.zeros((out_f, 16 - 2), jnp.float32)
    a2A = jnp.concatenate([a2_src.T, a2_dst.T, pad2], axis=1)
    a2B = jnp.concatenate([a2_dst.T, a2_src.T, pad2], axis=1)

    ids = jax.lax.broadcasted_iota(jnp.int32, (16, hdim), 0)
    cols = jax.lax.broadcasted_iota(jnp.int32, (16, hdim), 1)
    R16 = jnp.where(ids == cols // hid, 1.0, 0.0).astype(jnp.float32)
    R2 = jnp.where(jax.lax.broadcasted_iota(jnp.int32, (16, out_f), 0) == 0,
                   1.0, 0.0).astype(jnp.float32)

    blk = 1000
    grid = (n // blk,)

    def full(shape):
        return pl.BlockSpec(shape, lambda i: (0, 0))

    def rows(width):
        return pl.BlockSpec((blk, width), lambda i: (i, 0))

    h1, tabA, tabB = pl.pallas_call(
        _tc1_body,
        grid=grid,
        in_specs=[rows(f_in), full((f_in, hdim)), full((hdim, 16)),
                  full((hdim, 16))],
        out_specs=[rows(hdim), rows(16), rows(16)],
        out_shape=[jax.ShapeDtypeStruct((n, hdim), jnp.float32),
                   jax.ShapeDtypeStruct((n, 16), jnp.float32),
                   jax.ShapeDtypeStruct((n, 16), jnp.float32)],
    )(x, W1, acatA, acatB)

    n_pad = ((n + 8 * NSUB - 1) // (8 * NSUB)) * (8 * NSUB)
    zf = jnp.zeros((n_pad, hdim), jnp.float32)
    z16 = jnp.zeros((n_pad, 16), jnp.float32)
    sc1 = _make_sc_edge_kernel(n_pad, e, hdim, 200, heads)
    outp1, denp1 = sc1(src, dst, tabA, tabB, h1, zf, z16)

    h2, tab2A, tab2B = pl.pallas_call(
        _tc2_body,
        grid=grid,
        in_specs=[rows(hdim), rows(hdim), rows(16), rows(16),
                  full((16, hdim)), pl.BlockSpec((1, hdim), lambda i: (0, 0)),
                  full((hdim, out_f)), full((out_f, 16)), full((out_f, 16))],
        out_specs=[rows(out_f), rows(16), rows(16)],
        out_shape=[jax.ShapeDtypeStruct((n, out_f), jnp.float32),
                   jax.ShapeDtypeStruct((n, 16), jnp.float32),
                   jax.ShapeDtypeStruct((n, 16), jnp.float32)],
    )(outp1[:n], outp1[n_pad:n_pad + n], denp1[:n], denp1[n_pad:n_pad + n],
      R16, b1.reshape(1, hdim), W2, a2A, a2B)

    zf2 = jnp.zeros((n_pad, out_f), jnp.float32)
    sc2 = _make_sc_edge_kernel(n_pad, e, out_f, 200, 1)
    outp2, denp2 = sc2(src, dst, tab2A, tab2B, h2, zf2, z16)

    out2 = pl.pallas_call(
        _tc3_body,
        grid=grid,
        in_specs=[rows(out_f), rows(out_f), rows(16), rows(16),
                  full((16, out_f)), pl.BlockSpec((1, out_f), lambda i: (0, 0))],
        out_specs=rows(out_f),
        out_shape=jax.ShapeDtypeStruct((n, out_f), jnp.float32),
    )(outp2[:n], outp2[n_pad:n_pad + n], denp2[:n], denp2[n_pad:n_pad + n],
      R2, b2.reshape(1, out_f))

    take = _make_sc_take_kernel(n, out_f, nb)
    return take(out2, node_index)


# speedup vs baseline: 71.1499x; 1.5454x over previous
import functools

import jax
import jax.numpy as jnp
from jax import lax
from jax.experimental import pallas as pl
from jax.experimental.pallas import tpu as pltpu
from jax.experimental.pallas import tpu_sc as plsc

NCORE = 2
NSUB = 16
NW = NCORE * NSUB

_GDN = lax.GatherDimensionNumbers(
    offset_dims=(), collapsed_slice_dims=(0,), start_index_map=(0,))


def _lane_gather(v16, idx16):
    return lax.gather(v16, idx16[:, None], dimension_numbers=_GDN,
                      slice_sizes=(1,),
                      mode=lax.GatherScatterMode.PROMISE_IN_BOUNDS)


def _tc1_body(x_ref, w1_ref, acatA_ref, acatB_ref, h_ref, tA_ref, tB_ref):
    h = jnp.dot(x_ref[...], w1_ref[...], preferred_element_type=jnp.float32)
    h_ref[...] = h
    tA_ref[...] = jnp.dot(h, acatA_ref[...], preferred_element_type=jnp.float32)
    tB_ref[...] = jnp.dot(h, acatB_ref[...], preferred_element_type=jnp.float32)


def _tc2_body(op0_ref, op1_ref, dp0_ref, dp1_ref, r16_ref, b1_ref, w2_ref,
              a2A_ref, a2B_ref, h2_ref, t2A_ref, t2B_ref):
    den = jnp.dot(dp0_ref[...] + dp1_ref[...], r16_ref[...],
                  preferred_element_type=jnp.float32)
    out1 = (op0_ref[...] + op1_ref[...]) / (den + 1e-9) + b1_ref[...]
    hmid = jnp.where(out1 > 0, out1, jnp.exp(jnp.minimum(out1, 0.0)) - 1.0)
    h2 = jnp.dot(hmid, w2_ref[...], preferred_element_type=jnp.float32)
    h2_ref[...] = h2
    t2A_ref[...] = jnp.dot(h2, a2A_ref[...], preferred_element_type=jnp.float32)
    t2B_ref[...] = jnp.dot(h2, a2B_ref[...], preferred_element_type=jnp.float32)


def _tc3_body(mp0_ref, mp1_ref, ep0_ref, ep1_ref, r2_ref, b2_ref, out_ref):
    den = jnp.dot(ep0_ref[...] + ep1_ref[...], r2_ref[...],
                  preferred_element_type=jnp.float32)
    out_ref[...] = (mp0_ref[...] + mp1_ref[...]) / (den + 1e-9) + b2_ref[...]


def _make_sc_edge_kernel(n_pad, n_edges, feat, chunk, heads):
    ept = n_edges // NW
    nchunk = ept // chunk
    assert nchunk % 2 == 0
    rows_per_sub = n_pad // NSUB
    nheads_blk = feat // 16

    mesh = plsc.VectorSubcoreMesh(core_axis_name="c", subcore_axis_name="s")

    @functools.partial(
        pl.kernel,
        mesh=mesh,
        compiler_params=pltpu.CompilerParams(use_tc_tiling_on_sc=False),
        out_type=[
            jax.ShapeDtypeStruct((NCORE * n_pad, feat), jnp.float32),
            jax.ShapeDtypeStruct((NCORE * n_pad, 16), jnp.float32),
        ],
        scratch_types=[
            pltpu.VMEM((nchunk, chunk), jnp.int32),
            pltpu.VMEM((nchunk, chunk), jnp.int32),
            pltpu.VMEM((2, chunk, 16), jnp.float32),
            pltpu.VMEM((2, chunk, 16), jnp.float32),
            pltpu.VMEM((2, chunk, feat), jnp.float32),
            pltpu.VMEM((2, chunk, 16), jnp.float32),
            pltpu.VMEM_SHARED((n_pad, feat), jnp.float32),
            pltpu.VMEM_SHARED((n_pad, 16), jnp.float32),
            pltpu.SemaphoreType.DMA,
            pltpu.SemaphoreType.DMA,
            pltpu.SemaphoreType.DMA,
            pltpu.SemaphoreType.DMA,
        ],
    )
    def sc_kernel(src_hbm, dst_hbm, tabA_hbm, tabB_hbm, h_hbm, zf_hbm, z16_hbm,
                  outp_hbm, denp_hbm,
                  sidx, didx, tabs, tabd, hbuf, exb, out_sh, den_sh,
                  g0, g1, s0, s1):
        c = lax.axis_index("c")
        s = lax.axis_index("s")
        wid = c * NSUB + s
        rbase = s * rows_per_sub
        gsem = (g0, g1)
        ssem = (s0, s1)

        pltpu.sync_copy(zf_hbm.at[pl.ds(rbase, rows_per_sub)],
                        out_sh.at[pl.ds(rbase, rows_per_sub)])
        pltpu.sync_copy(z16_hbm.at[pl.ds(rbase, rows_per_sub)],
                        den_sh.at[pl.ds(rbase, rows_per_sub)])
        plsc.subcore_barrier()

        pltpu.sync_copy(src_hbm.at[pl.ds(wid * nchunk, nchunk)], sidx)
        pltpu.sync_copy(dst_hbm.at[pl.ds(wid * nchunk, nchunk)], didx)

        def issue_gathers(k, p):
            pltpu.async_copy(tabA_hbm.at[sidx.at[k]], tabs.at[p], gsem[p])
            pltpu.async_copy(tabB_hbm.at[didx.at[k]], tabd.at[p], gsem[p])
            pltpu.async_copy(h_hbm.at[sidx.at[k]], hbuf.at[p], gsem[p])

        def wait_gathers(k, p):
            pltpu.make_async_copy(tabA_hbm.at[sidx.at[k]], tabs.at[p],
                                  gsem[p]).wait()
            pltpu.make_async_copy(tabB_hbm.at[didx.at[k]], tabd.at[p],
                                  gsem[p]).wait()
            pltpu.make_async_copy(h_hbm.at[sidx.at[k]], hbuf.at[p],
                                  gsem[p]).wait()

        def issue_scatters(k, p):
            pltpu.async_copy(exb.at[p], den_sh.at[didx.at[k]], ssem[p],
                             add=True)
            pltpu.async_copy(hbuf.at[p], out_sh.at[didx.at[k]], ssem[p],
                             add=True)

        def wait_scatters(k, p):
            pltpu.make_async_copy(exb.at[p], den_sh.at[didx.at[k]],
                                  ssem[p]).wait()
            pltpu.make_async_copy(hbuf.at[p], out_sh.at[didx.at[k]],
                                  ssem[p]).wait()

        def compute(p):
            tabs_p, tabd_p, hbuf_p, exb_p = (tabs.at[p], tabd.at[p],
                                             hbuf.at[p], exb.at[p])

            @pl.loop(0, chunk)
            def _edge(j):
                ea = tabs_p[j, :]
                eb = tabd_p[j, :]
                e = ea + eb
                e = jnp.maximum(e, 0.2 * e)
                ex = jnp.exp(e)
                exb_p[j, :] = ex
                for h in range(heads):
                    idx = jnp.full((16,), h, dtype=jnp.int32)
                    bc = _lane_gather(ex, idx)
                    if heads == 1:
                        for fblk in range(nheads_blk):
                            slc = (j, pl.ds(fblk * 16, 16))
                            hbuf_p[slc] = hbuf_p[slc] * bc
                    else:
                        slc = (j, pl.ds(h * 16, 16))
                        hbuf_p[slc] = hbuf_p[slc] * bc

        issue_gathers(0, 0)

        @pl.loop(0, nchunk // 2)
        def _pair(i):
            for p in (0, 1):
                kk = 2 * i + p
                wait_gathers(kk, p)

                @pl.when(kk + 1 < nchunk)
                def _prefetch():
                    @pl.when(kk >= 1)
                    def _drain():
                        wait_scatters(kk - 1, 1 - p)
                    issue_gathers(kk + 1, 1 - p)

                compute(p)
                issue_scatters(kk, p)

        wait_scatters(nchunk - 2, 0)
        wait_scatters(nchunk - 1, 1)
        plsc.subcore_barrier()
        pltpu.sync_copy(out_sh.at[pl.ds(rbase, rows_per_sub)],
                        outp_hbm.at[pl.ds(c * n_pad + rbase, rows_per_sub)])
        pltpu.sync_copy(den_sh.at[pl.ds(rbase, rows_per_sub)],
                        denp_hbm.at[pl.ds(c * n_pad + rbase, rows_per_sub)])

    return sc_kernel


def _make_sc_take_kernel(n_nodes, feat, nb):
    rows_per_tile = nb // NW
    mesh = plsc.VectorSubcoreMesh(core_axis_name="c", subcore_axis_name="s")

    @functools.partial(
        pl.kernel,
        mesh=mesh,
        compiler_params=pltpu.CompilerParams(use_tc_tiling_on_sc=False),
        out_type=jax.ShapeDtypeStruct((nb, feat), jnp.float32),
        scratch_types=[
            pltpu.VMEM((rows_per_tile,), jnp.int32),
            pltpu.VMEM((rows_per_tile, feat), jnp.float32),
        ],
    )
    def take_kernel(tab_hbm, idx_hbm, out_hbm, idxv, rowsv):
        c = lax.axis_index("c")
        s = lax.axis_index("s")
        wid = c * NSUB + s
        base = wid * rows_per_tile
        pltpu.sync_copy(idx_hbm.at[pl.ds(base, rows_per_tile)], idxv)
        pltpu.sync_copy(tab_hbm.at[idxv], rowsv)
        pltpu.sync_copy(rowsv, out_hbm.at[pl.ds(base, rows_per_tile)])

    return take_kernel


@jax.jit
def kernel(x, edge_index, node_index, W1, a1_src, a1_dst, b1,
           W2, a2_src, a2_dst, b2):
    n, f_in = x.shape
    e = edge_index.shape[1]
    heads, hid = a1_src.shape
    out_f = W2.shape[1]
    nb = node_index.shape[0]
    hdim = heads * hid

    src = edge_index[0]
    dst = edge_index[1]

    eyeh = jnp.eye(heads, dtype=jnp.float32)
    A1s = (eyeh[:, None, :] * a1_src[:, :, None]).reshape(hdim, heads)
    A1d = (eyeh[:, None, :] * a1_dst[:, :, None]).reshape(hdim, heads)
    acatA = jnp.concatenate([A1s, A1d], axis=1)
    acatB = jnp.concatenate([A1d, A1s], axis=1)
    pad2 = jnp.zeros((out_f, 16 - 2), jnp.float32)
    a2A = jnp.concatenate([a2_src.T, a2_dst.T, pad2], axis=1)
    a2B = jnp.concatenate([a2_dst.T, a2_src.T, pad2], axis=1)

    ids = jax.lax.broadcasted_iota(jnp.int32, (16, hdim), 0)
    cols = jax.lax.broadcasted_iota(jnp.int32, (16, hdim), 1)
    R16 = jnp.where(ids == cols // hid, 1.0, 0.0).astype(jnp.float32)
    R2 = jnp.where(jax.lax.broadcasted_iota(jnp.int32, (16, out_f), 0) == 0,
                   1.0, 0.0).astype(jnp.float32)

    blk = 1000
    grid = (n // blk,)

    def full(shape):
        return pl.BlockSpec(shape, lambda i: (0, 0))

    def rows(width):
        return pl.BlockSpec((blk, width), lambda i: (i, 0))

    h1, tabA, tabB = pl.pallas_call(
        _tc1_body,
        grid=grid,
        in_specs=[rows(f_in), full((f_in, hdim)), full((hdim, 16)),
                  full((hdim, 16))],
        out_specs=[rows(hdim), rows(16), rows(16)],
        out_shape=[jax.ShapeDtypeStruct((n, hdim), jnp.float32),
                   jax.ShapeDtypeStruct((n, 16), jnp.float32),
                   jax.ShapeDtypeStruct((n, 16), jnp.float32)],
    )(x, W1, acatA, acatB)

    n_pad = ((n + 8 * NSUB - 1) // (8 * NSUB)) * (8 * NSUB)
    zf = jnp.zeros((n_pad, hdim), jnp.float32)
    z16 = jnp.zeros((n_pad, 16), jnp.float32)
    sc1 = _make_sc_edge_kernel(n_pad, e, hdim, 40, heads)
    outp1, denp1 = sc1(src.reshape(-1, 40), dst.reshape(-1, 40),
                       tabA, tabB, h1, zf, z16)

    h2, tab2A, tab2B = pl.pallas_call(
        _tc2_body,
        grid=grid,
        in_specs=[rows(hdim), rows(hdim), rows(16), rows(16),
                  full((16, hdim)), pl.BlockSpec((1, hdim), lambda i: (0, 0)),
                  full((hdim, out_f)), full((out_f, 16)), full((out_f, 16))],
        out_specs=[rows(out_f), rows(16), rows(16)],
        out_shape=[jax.ShapeDtypeStruct((n, out_f), jnp.float32),
                   jax.ShapeDtypeStruct((n, 16), jnp.float32),
                   jax.ShapeDtypeStruct((n, 16), jnp.float32)],
    )(outp1[:n], outp1[n_pad:n_pad + n], denp1[:n], denp1[n_pad:n_pad + n],
      R16, b1.reshape(1, hdim), W2, a2A, a2B)

    zf2 = jnp.zeros((n_pad, out_f), jnp.float32)
    sc2 = _make_sc_edge_kernel(n_pad, e, out_f, 200, 1)
    outp2, denp2 = sc2(src.reshape(-1, 200), dst.reshape(-1, 200),
                       tab2A, tab2B, h2, zf2, z16)

    out2 = pl.pallas_call(
        _tc3_body,
        grid=grid,
        in_specs=[rows(out_f), rows(out_f), rows(16), rows(16),
                  full((16, out_f)), pl.BlockSpec((1, out_f), lambda i: (0, 0))],
        out_specs=rows(out_f),
        out_shape=jax.ShapeDtypeStruct((n, out_f), jnp.float32),
    )(outp2[:n], outp2[n_pad:n_pad + n], denp2[:n], denp2[n_pad:n_pad + n],
      R2, b2.reshape(1, out_f))

    take = _make_sc_take_kernel(n, out_f, nb)
    return take(out2, node_index)
